# Initial kernel scaffold; baseline (speedup 1.0000x reference)
import jax
import jax.numpy as jnp
from jax.experimental import pallas as pl


def kernel(x, edge_index, edge_attr, Wl1, Wr1, b1, Wlh, Wrh, bh, Wl2, Wr2, b2):
    raise NotImplementedError("write your pallas kernel here")


# speedup vs baseline: 6.1399x; 6.1399x over previous
import functools

import jax
import jax.numpy as jnp
from jax import lax
from jax.experimental import pallas as pl
from jax.experimental.pallas import tpu as pltpu
from jax.experimental.pallas import tpu_sc as plsc

N = 10000
E = 320000
NC, NS = 2, 16
NW = NC * NS
K = 80
EPC = E // NS
CHC = EPC // K
EPW = E // NW
CHW = EPW // K
NP8 = 10008
RPS = 632
R = 1000
G = N // R


def _stripe(s):
    return pl.multiple_of(jnp.minimum(s * RPS, NP8 - RPS), 8)


@functools.cache
def _make_colsplit(with_deg):
    mesh = plsc.VectorSubcoreMesh(core_axis_name="c", subcore_axis_name="s",
                                  num_cores=NC, num_subcores=NS)
    out_type = [jax.ShapeDtypeStruct((NC, NP8, 64), jnp.float32)]
    scratch = [
        pltpu.VMEM((CHC, K), jnp.int32),
        pltpu.VMEM((CHC, K), jnp.int32),
        pltpu.VMEM((K, 64), jnp.float32),
        pltpu.VMEM_SHARED((NP8, 64), jnp.float32),
        pltpu.SemaphoreType.DMA,
    ]
    if with_deg:
        out_type.append(jax.ShapeDtypeStruct((NP8, 16), jnp.float32))
        scratch += [
            pltpu.VMEM((K, 16), jnp.float32),
            pltpu.VMEM_SHARED((NP8, 16), jnp.float32),
        ]

    def body(h_hbm, src_hbm, dst_hbm, z_hbm, z16_hbm, *rest):
        if with_deg:
            (out_hbm, deg_hbm, srcb, dstb, rows, agg_sh, sem,
             onesb, deg_sh) = rest
        else:
            (out_hbm, srcb, dstb, rows, agg_sh, sem) = rest
        c = lax.axis_index("c")
        s = lax.axis_index("s")
        pltpu.sync_copy(src_hbm.at[s], srcb)
        pltpu.sync_copy(dst_hbm.at[s], dstb)
        r0 = _stripe(s)
        pltpu.sync_copy(z_hbm.at[pl.ds(r0, RPS)], agg_sh.at[pl.ds(r0, RPS)])
        if with_deg:
            @pl.when(c == 0)
            def _():
                pltpu.sync_copy(z16_hbm.at[pl.ds(r0, RPS)],
                                deg_sh.at[pl.ds(r0, RPS)])
            for r in range(K):
                onesb[r, :] = jnp.ones((16,), jnp.float32)
        plsc.subcore_barrier()

        def chunk(g, carry):
            pltpu.async_copy(h_hbm.at[c].at[srcb.at[g]], rows, sem).wait()
            pltpu.sync_copy(rows, agg_sh.at[dstb.at[g]], add=True)
            if with_deg:
                @pl.when(c == 0)
                def _():
                    pltpu.sync_copy(onesb, deg_sh.at[dstb.at[g]], add=True)
            return carry

        lax.fori_loop(0, CHC, chunk, 0)
        plsc.subcore_barrier()
        pltpu.sync_copy(agg_sh.at[pl.ds(r0, RPS)],
                        out_hbm.at[c, pl.ds(r0, RPS)])
        if with_deg:
            @pl.when(c == 0)
            def _():
                pltpu.sync_copy(deg_sh.at[pl.ds(r0, RPS)],
                                deg_hbm.at[pl.ds(r0, RPS)])

    return pl.kernel(body, out_type=tuple(out_type), mesh=mesh,
                     scratch_types=scratch,
                     compiler_params=pltpu.CompilerParams(
                         use_tc_tiling_on_sc=False))


@functools.cache
def _make_edgesplit():
    mesh = plsc.VectorSubcoreMesh(core_axis_name="c", subcore_axis_name="s",
                                  num_cores=NC, num_subcores=NS)
    scratch = [
        pltpu.VMEM((CHW, K), jnp.int32),
        pltpu.VMEM((CHW, K), jnp.int32),
        pltpu.VMEM((K, 64), jnp.float32),
        pltpu.VMEM_SHARED((NP8, 64), jnp.float32),
        pltpu.SemaphoreType.DMA,
    ]

    def body(h_hbm, src_hbm, dst_hbm, z_hbm, out_hbm,
             srcb, dstb, rows, agg_sh, sem):
        c = lax.axis_index("c")
        s = lax.axis_index("s")
        wid = c * NS + s
        pltpu.sync_copy(src_hbm.at[wid], srcb)
        pltpu.sync_copy(dst_hbm.at[wid], dstb)
        r0 = _stripe(s)
        pltpu.sync_copy(z_hbm.at[pl.ds(r0, RPS)], agg_sh.at[pl.ds(r0, RPS)])
        plsc.subcore_barrier()

        def chunk(g, carry):
            pltpu.async_copy(h_hbm.at[srcb.at[g]], rows, sem).wait()
            pltpu.sync_copy(rows, agg_sh.at[dstb.at[g]], add=True)
            return carry

        lax.fori_loop(0, CHW, chunk, 0)
        plsc.subcore_barrier()
        pltpu.sync_copy(agg_sh.at[pl.ds(r0, RPS)],
                        out_hbm.at[c, pl.ds(r0, RPS)])

    return pl.kernel(
        body, out_type=jax.ShapeDtypeStruct((NC, NP8, 64), jnp.float32),
        mesh=mesh, scratch_types=scratch,
        compiler_params=pltpu.CompilerParams(use_tc_tiling_on_sc=False))


def _full(shape):
    return pl.BlockSpec(shape, lambda i: tuple(0 for _ in shape))


def _split_cols(res, hl_ref):
    hl_ref[0] = res[:, :64]
    hl_ref[1] = res[:, 64:]


def _tc_transform(x, wlT, wrT, b):
    d_in = wlT.shape[0]

    def body(x_ref, wl_ref, wr_ref, b_ref, hl_ref, sf_ref):
        xb = x_ref[...]
        _split_cols(jnp.dot(xb, wl_ref[...],
                            preferred_element_type=jnp.float32), hl_ref)
        sf_ref[...] = jnp.dot(xb, wr_ref[...],
                              preferred_element_type=jnp.float32) + b_ref[...]

    return pl.pallas_call(
        body,
        grid=(G,),
        in_specs=[
            pl.BlockSpec((R, d_in), lambda i: (i, 0)),
            _full((d_in, 128)), _full((d_in, 128)), _full((1, 128)),
        ],
        out_specs=[pl.BlockSpec((2, R, 64), lambda i: (0, i, 0)),
                   pl.BlockSpec((R, 128), lambda i: (i, 0))],
        out_shape=[jax.ShapeDtypeStruct((2, N, 64), jnp.float32),
                   jax.ShapeDtypeStruct((N, 128), jnp.float32)],
    )(x, wlT, wrT, b)


def _inv_deg(deg_ref):
    return 1.0 / jnp.maximum(deg_ref[:, 0:1], 1.0)


def _tc_mid(aggp, deg, sf, wlT, wrT, b, act, split_out):
    d_out = wlT.shape[1]

    def body(aggp_ref, deg_ref, sf_ref, wl_ref, wr_ref, b_ref,
             hl_ref, sfo_ref):
        agg = jnp.concatenate([aggp_ref[0], aggp_ref[1]], axis=1)
        z = agg * _inv_deg(deg_ref) + sf_ref[...]
        if act == "relu":
            h = jnp.maximum(z, 0.0)
        else:
            h = jnp.where(z > 0, z, jnp.exp(jnp.minimum(z, 0.0)) - 1.0)
        hl = jnp.dot(h, wl_ref[...], preferred_element_type=jnp.float32)
        if split_out:
            _split_cols(hl, hl_ref)
        else:
            hl_ref[...] = hl
        sfo_ref[...] = jnp.dot(h, wr_ref[...],
                               preferred_element_type=jnp.float32) + b_ref[...]

    if split_out:
        hl_spec = pl.BlockSpec((2, R, 64), lambda i: (0, i, 0))
        hl_shape = jax.ShapeDtypeStruct((2, N, 64), jnp.float32)
    else:
        hl_spec = pl.BlockSpec((R, d_out), lambda i: (i, 0))
        hl_shape = jax.ShapeDtypeStruct((N, d_out), jnp.float32)

    return pl.pallas_call(
        body,
        grid=(G,),
        in_specs=[
            pl.BlockSpec((NC, R, 64), lambda i: (0, i, 0)),
            pl.BlockSpec((R, 16), lambda i: (i, 0)),
            pl.BlockSpec((R, 128), lambda i: (i, 0)),
            _full((128, d_out)), _full((128, d_out)), _full((1, d_out)),
        ],
        out_specs=[hl_spec, pl.BlockSpec((R, d_out), lambda i: (i, 0))],
        out_shape=[hl_shape, jax.ShapeDtypeStruct((N, d_out), jnp.float32)],
    )(aggp, deg, sf, wlT, wrT, b)


def _tc_final(aggp, deg, sf):
    def body(aggp_ref, deg_ref, sf_ref, out_ref):
        agg = aggp_ref[0] + aggp_ref[1]
        z = agg * _inv_deg(deg_ref) + sf_ref[...]
        m = jnp.max(z, axis=1, keepdims=True)
        lse = m + jnp.log(jnp.sum(jnp.exp(z - m), axis=1, keepdims=True))
        out_ref[...] = z - lse

    return pl.pallas_call(
        body,
        grid=(G,),
        in_specs=[
            pl.BlockSpec((NC, R, 64), lambda i: (0, i, 0)),
            pl.BlockSpec((R, 16), lambda i: (i, 0)),
            pl.BlockSpec((R, 64), lambda i: (i, 0)),
        ],
        out_specs=pl.BlockSpec((R, 64), lambda i: (i, 0)),
        out_shape=jax.ShapeDtypeStruct((N, 64), jnp.float32),
    )(aggp, deg, sf)


def kernel(x, edge_index, edge_attr, Wl1, Wr1, b1, Wlh, Wrh, bh, Wl2, Wr2, b2):
    del edge_attr
    src_c = edge_index[0].reshape(NS, CHC, K)
    dst_c = edge_index[1].reshape(NS, CHC, K)
    src_w = edge_index[0].reshape(NW, CHW, K)
    dst_w = edge_index[1].reshape(NW, CHW, K)
    z64 = jnp.zeros((NP8, 64), jnp.float32)
    z16 = jnp.zeros((NP8, 16), jnp.float32)

    hl1, sf1 = _tc_transform(x, Wl1.T, Wr1.T, b1.reshape(1, -1))
    aggp1, deg = _make_colsplit(True)(hl1, src_c, dst_c, z64, z16)
    hl2, sf2 = _tc_mid(aggp1, deg, sf1, Wlh.T, Wrh.T, bh.reshape(1, -1),
                       "relu", True)
    aggp2 = _make_colsplit(False)(hl2, src_c, dst_c, z64, z16)
    if isinstance(aggp2, (tuple, list)):
        aggp2 = aggp2[0]
    hl3, sf3 = _tc_mid(aggp2, deg, sf2, Wl2.T, Wr2.T, b2.reshape(1, -1),
                       "elu", False)
    aggp3 = _make_edgesplit()(hl3, src_w, dst_w, z64)
    return _tc_final(aggp3, deg, sf3)


# speedup vs baseline: 11.0374x; 1.7976x over previous
import functools

import jax
import jax.numpy as jnp
from jax import lax
from jax.experimental import pallas as pl
from jax.experimental.pallas import tpu as pltpu
from jax.experimental.pallas import tpu_sc as plsc

N = 10000
E = 320000
NC, NS = 2, 16
NW = NC * NS
K = 80
EPC = E // NS
CHC = EPC // K
EPW = E // NW
CHW = EPW // K
NP8 = 10008
RPS = 632
NB = 4
LA = 2
R = 1000
G = N // R


def _stripe(s):
    return pl.multiple_of(jnp.minimum(s * RPS, NP8 - RPS), 8)


@functools.cache
def _make_colsplit(with_deg):
    mesh = plsc.VectorSubcoreMesh(core_axis_name="c", subcore_axis_name="s",
                                  num_cores=NC, num_subcores=NS)
    out_type = [jax.ShapeDtypeStruct((NC, NP8, 64), jnp.float32)]
    scratch = [
        pltpu.VMEM((CHC, K), jnp.int32),
        pltpu.VMEM((CHC, K), jnp.int32),
        [pltpu.VMEM((K, 64), jnp.float32)] * NB,
        pltpu.VMEM_SHARED((NP8, 64), jnp.float32),
        [pltpu.SemaphoreType.DMA] * NB,
        [pltpu.SemaphoreType.DMA] * NB,
        pltpu.SemaphoreType.DMA,
    ]
    if with_deg:
        out_type.append(jax.ShapeDtypeStruct((NP8, 16), jnp.float32))
        scratch += [
            pltpu.VMEM((K, 16), jnp.float32),
            pltpu.VMEM_SHARED((NP8, 16), jnp.float32),
        ]

    def body(h_hbm, src_hbm, dst_hbm, z_hbm, z16_hbm, *rest):
        if with_deg:
            (out_hbm, deg_hbm, srcb, dstb, rows, agg_sh, gsem, ssem, dsem,
             onesb, deg_sh) = rest
        else:
            (out_hbm, srcb, dstb, rows, agg_sh, gsem, ssem, dsem) = rest
        c = lax.axis_index("c")
        s = lax.axis_index("s")
        pltpu.sync_copy(src_hbm.at[s], srcb)
        pltpu.sync_copy(dst_hbm.at[s], dstb)
        r0 = _stripe(s)
        pltpu.sync_copy(z_hbm.at[pl.ds(r0, RPS)], agg_sh.at[pl.ds(r0, RPS)])
        if with_deg:
            @pl.when(c == 0)
            def _():
                pltpu.sync_copy(z16_hbm.at[pl.ds(r0, RPS)],
                                deg_sh.at[pl.ds(r0, RPS)])
            for r in range(K):
                onesb[r, :] = jnp.ones((16,), jnp.float32)
        h_view = h_hbm.at[c]
        for g in range(LA):
            pltpu.async_copy(h_view.at[srcb.at[g]], rows[g], gsem[g])
        plsc.subcore_barrier()

        def step(g, b):
            pltpu.make_async_copy(h_view.at[srcb.at[g]],
                                  rows[b], gsem[b]).wait()
            pltpu.async_copy(rows[b], agg_sh.at[dstb.at[g]], ssem[b],
                             add=True)
            if with_deg:
                @pl.when(c == 0)
                def _():
                    pltpu.async_copy(onesb, deg_sh.at[dstb.at[g]], dsem,
                                     add=True)
                    @pl.when(g >= LA)
                    def _():
                        pltpu.make_async_copy(
                            onesb, deg_sh.at[dstb.at[g]], dsem).wait()
            bs = (b - LA) % NB
            @pl.when(g >= LA)
            def _():
                pltpu.make_async_copy(rows[bs], agg_sh.at[dstb.at[g]],
                                      ssem[bs]).wait()
            bg = (b + LA) % NB
            @pl.when(g + LA < CHC)
            def _():
                pltpu.async_copy(h_view.at[srcb.at[jnp.minimum(
                    g + LA, CHC - 1)]], rows[bg], gsem[bg])

        T = (CHC // NB) * NB

        def outer(t, carry):
            for b in range(NB):
                step(t * NB + b, b)
            return carry

        lax.fori_loop(0, T // NB, outer, 0)
        for g in range(T, CHC):
            step(g, g % NB)
        for g in range(CHC - LA, CHC):
            pltpu.make_async_copy(rows[g % NB], agg_sh.at[dstb.at[0]],
                                  ssem[g % NB]).wait()
            if with_deg:
                @pl.when(c == 0)
                def _():
                    pltpu.make_async_copy(onesb, deg_sh.at[dstb.at[0]],
                                          dsem).wait()
        plsc.subcore_barrier()
        pltpu.sync_copy(agg_sh.at[pl.ds(r0, RPS)],
                        out_hbm.at[c, pl.ds(r0, RPS)])
        if with_deg:
            @pl.when(c == 0)
            def _():
                pltpu.sync_copy(deg_sh.at[pl.ds(r0, RPS)],
                                deg_hbm.at[pl.ds(r0, RPS)])

    return pl.kernel(body, out_type=tuple(out_type), mesh=mesh,
                     scratch_types=scratch,
                     compiler_params=pltpu.CompilerParams(
                         use_tc_tiling_on_sc=False))


@functools.cache
def _make_edgesplit():
    mesh = plsc.VectorSubcoreMesh(core_axis_name="c", subcore_axis_name="s",
                                  num_cores=NC, num_subcores=NS)
    scratch = [
        pltpu.VMEM((CHW, K), jnp.int32),
        pltpu.VMEM((CHW, K), jnp.int32),
        [pltpu.VMEM((K, 64), jnp.float32)] * NB,
        pltpu.VMEM_SHARED((NP8, 64), jnp.float32),
        [pltpu.SemaphoreType.DMA] * NB,
        [pltpu.SemaphoreType.DMA] * NB,
    ]

    def body(h_hbm, src_hbm, dst_hbm, z_hbm, out_hbm,
             srcb, dstb, rows, agg_sh, gsem, ssem):
        c = lax.axis_index("c")
        s = lax.axis_index("s")
        wid = c * NS + s
        pltpu.sync_copy(src_hbm.at[wid], srcb)
        pltpu.sync_copy(dst_hbm.at[wid], dstb)
        r0 = _stripe(s)
        pltpu.sync_copy(z_hbm.at[pl.ds(r0, RPS)], agg_sh.at[pl.ds(r0, RPS)])
        for g in range(LA):
            pltpu.async_copy(h_hbm.at[srcb.at[g]], rows[g], gsem[g])
        plsc.subcore_barrier()

        def step(g, b):
            pltpu.make_async_copy(h_hbm.at[srcb.at[g]],
                                  rows[b], gsem[b]).wait()
            pltpu.async_copy(rows[b], agg_sh.at[dstb.at[g]], ssem[b],
                             add=True)
            bs = (b - LA) % NB
            @pl.when(g >= LA)
            def _():
                pltpu.make_async_copy(rows[bs], agg_sh.at[dstb.at[g]],
                                      ssem[bs]).wait()
            bg = (b + LA) % NB
            @pl.when(g + LA < CHW)
            def _():
                pltpu.async_copy(h_hbm.at[srcb.at[jnp.minimum(
                    g + LA, CHW - 1)]], rows[bg], gsem[bg])

        T = (CHW // NB) * NB

        def outer(t, carry):
            for b in range(NB):
                step(t * NB + b, b)
            return carry

        lax.fori_loop(0, T // NB, outer, 0)
        for g in range(T, CHW):
            step(g, g % NB)
        for g in range(CHW - LA, CHW):
            pltpu.make_async_copy(rows[g % NB], agg_sh.at[dstb.at[0]],
                                  ssem[g % NB]).wait()
        plsc.subcore_barrier()
        pltpu.sync_copy(agg_sh.at[pl.ds(r0, RPS)],
                        out_hbm.at[c, pl.ds(r0, RPS)])

    return pl.kernel(
        body, out_type=jax.ShapeDtypeStruct((NC, NP8, 64), jnp.float32),
        mesh=mesh, scratch_types=scratch,
        compiler_params=pltpu.CompilerParams(use_tc_tiling_on_sc=False))


def _full(shape):
    return pl.BlockSpec(shape, lambda i: tuple(0 for _ in shape))


def _split_cols(res, hl_ref):
    hl_ref[0] = res[:, :64]
    hl_ref[1] = res[:, 64:]


def _tc_transform(x, wlT, wrT, b):
    d_in = wlT.shape[0]

    def body(x_ref, wl_ref, wr_ref, b_ref, hl_ref, sf_ref):
        xb = x_ref[...]
        _split_cols(jnp.dot(xb, wl_ref[...],
                            preferred_element_type=jnp.float32), hl_ref)
        sf_ref[...] = jnp.dot(xb, wr_ref[...],
                              preferred_element_type=jnp.float32) + b_ref[...]

    return pl.pallas_call(
        body,
        grid=(G,),
        in_specs=[
            pl.BlockSpec((R, d_in), lambda i: (i, 0)),
            _full((d_in, 128)), _full((d_in, 128)), _full((1, 128)),
        ],
        out_specs=[pl.BlockSpec((2, R, 64), lambda i: (0, i, 0)),
                   pl.BlockSpec((R, 128), lambda i: (i, 0))],
        out_shape=[jax.ShapeDtypeStruct((2, N, 64), jnp.float32),
                   jax.ShapeDtypeStruct((N, 128), jnp.float32)],
    )(x, wlT, wrT, b)


def _inv_deg(deg_ref):
    return 1.0 / jnp.maximum(deg_ref[:, 0:1], 1.0)


def _tc_mid(aggp, deg, sf, wlT, wrT, b, act, split_out):
    d_out = wlT.shape[1]

    def body(aggp_ref, deg_ref, sf_ref, wl_ref, wr_ref, b_ref,
             hl_ref, sfo_ref):
        agg = jnp.concatenate([aggp_ref[0], aggp_ref[1]], axis=1)
        z = agg * _inv_deg(deg_ref) + sf_ref[...]
        if act == "relu":
            h = jnp.maximum(z, 0.0)
        else:
            h = jnp.where(z > 0, z, jnp.exp(jnp.minimum(z, 0.0)) - 1.0)
        hl = jnp.dot(h, wl_ref[...], preferred_element_type=jnp.float32)
        if split_out:
            _split_cols(hl, hl_ref)
        else:
            hl_ref[...] = hl
        sfo_ref[...] = jnp.dot(h, wr_ref[...],
                               preferred_element_type=jnp.float32) + b_ref[...]

    if split_out:
        hl_spec = pl.BlockSpec((2, R, 64), lambda i: (0, i, 0))
        hl_shape = jax.ShapeDtypeStruct((2, N, 64), jnp.float32)
    else:
        hl_spec = pl.BlockSpec((R, d_out), lambda i: (i, 0))
        hl_shape = jax.ShapeDtypeStruct((N, d_out), jnp.float32)

    return pl.pallas_call(
        body,
        grid=(G,),
        in_specs=[
            pl.BlockSpec((NC, R, 64), lambda i: (0, i, 0)),
            pl.BlockSpec((R, 16), lambda i: (i, 0)),
            pl.BlockSpec((R, 128), lambda i: (i, 0)),
            _full((128, d_out)), _full((128, d_out)), _full((1, d_out)),
        ],
        out_specs=[hl_spec, pl.BlockSpec((R, d_out), lambda i: (i, 0))],
        out_shape=[hl_shape, jax.ShapeDtypeStruct((N, d_out), jnp.float32)],
    )(aggp, deg, sf, wlT, wrT, b)


def _tc_final(aggp, deg, sf):
    def body(aggp_ref, deg_ref, sf_ref, out_ref):
        agg = aggp_ref[0] + aggp_ref[1]
        z = agg * _inv_deg(deg_ref) + sf_ref[...]
        m = jnp.max(z, axis=1, keepdims=True)
        lse = m + jnp.log(jnp.sum(jnp.exp(z - m), axis=1, keepdims=True))
        out_ref[...] = z - lse

    return pl.pallas_call(
        body,
        grid=(G,),
        in_specs=[
            pl.BlockSpec((NC, R, 64), lambda i: (0, i, 0)),
            pl.BlockSpec((R, 16), lambda i: (i, 0)),
            pl.BlockSpec((R, 64), lambda i: (i, 0)),
        ],
        out_specs=pl.BlockSpec((R, 64), lambda i: (i, 0)),
        out_shape=jax.ShapeDtypeStruct((N, 64), jnp.float32),
    )(aggp, deg, sf)


def kernel(x, edge_index, edge_attr, Wl1, Wr1, b1, Wlh, Wrh, bh, Wl2, Wr2, b2):
    del edge_attr
    src_c = edge_index[0].reshape(NS, CHC, K)
    dst_c = edge_index[1].reshape(NS, CHC, K)
    src_w = edge_index[0].reshape(NW, CHW, K)
    dst_w = edge_index[1].reshape(NW, CHW, K)
    z64 = jnp.zeros((NP8, 64), jnp.float32)
    z16 = jnp.zeros((NP8, 16), jnp.float32)

    hl1, sf1 = _tc_transform(x, Wl1.T, Wr1.T, b1.reshape(1, -1))
    aggp1, deg = _make_colsplit(True)(hl1, src_c, dst_c, z64, z16)
    hl2, sf2 = _tc_mid(aggp1, deg, sf1, Wlh.T, Wrh.T, bh.reshape(1, -1),
                       "relu", True)
    aggp2 = _make_colsplit(False)(hl2, src_c, dst_c, z64, z16)
    if isinstance(aggp2, (tuple, list)):
        aggp2 = aggp2[0]
    hl3, sf3 = _tc_mid(aggp2, deg, sf2, Wl2.T, Wr2.T, b2.reshape(1, -1),
                       "elu", False)
    aggp3 = _make_edgesplit()(hl3, src_w, dst_w, z64)
    return _tc_final(aggp3, deg, sf3)


# speedup vs baseline: 11.7493x; 1.0645x over previous
import functools

import jax
import jax.numpy as jnp
from jax import lax
from jax.experimental import pallas as pl
from jax.experimental.pallas import tpu as pltpu
from jax.experimental.pallas import tpu_sc as plsc

N = 10000
E = 320000
NC, NS = 2, 16
NW = NC * NS
K = 80
EPC = E // NS
CHC = EPC // K
EPW = E // NW
CHW = EPW // K
NP8 = 10008
RPS = 632
NB = 6
LA = 3
R = 1000
G = N // R


def _stripe(s):
    return pl.multiple_of(jnp.minimum(s * RPS, NP8 - RPS), 8)


@functools.cache
def _make_colsplit(DH, with_deg):
    mesh = plsc.VectorSubcoreMesh(core_axis_name="c", subcore_axis_name="s",
                                  num_cores=NC, num_subcores=NS)
    out_type = [jax.ShapeDtypeStruct((NC, NP8, DH), jnp.float32)]
    scratch = [
        pltpu.VMEM((CHC, K), jnp.int32),
        pltpu.VMEM((CHC, K), jnp.int32),
        [pltpu.VMEM((K, DH), jnp.float32)] * NB,
        pltpu.VMEM_SHARED((NP8, DH), jnp.float32),
        [pltpu.SemaphoreType.DMA] * NB,
        [pltpu.SemaphoreType.DMA] * NB,
        pltpu.SemaphoreType.DMA,
    ]
    if with_deg:
        out_type.append(jax.ShapeDtypeStruct((NP8, 16), jnp.float32))
        scratch += [
            pltpu.VMEM((K, 16), jnp.float32),
            pltpu.VMEM_SHARED((NP8, 16), jnp.float32),
        ]

    def body(h_hbm, src_hbm, dst_hbm, z_hbm, z16_hbm, *rest):
        if with_deg:
            (out_hbm, deg_hbm, srcb, dstb, rows, agg_sh, gsem, ssem, dsem,
             onesb, deg_sh) = rest
        else:
            (out_hbm, srcb, dstb, rows, agg_sh, gsem, ssem, dsem) = rest
        c = lax.axis_index("c")
        s = lax.axis_index("s")
        pltpu.sync_copy(src_hbm.at[s], srcb)
        pltpu.sync_copy(dst_hbm.at[s], dstb)
        r0 = _stripe(s)
        pltpu.sync_copy(z_hbm.at[pl.ds(r0, RPS)], agg_sh.at[pl.ds(r0, RPS)])
        if with_deg:
            @pl.when(c == 0)
            def _():
                pltpu.sync_copy(z16_hbm.at[pl.ds(r0, RPS)],
                                deg_sh.at[pl.ds(r0, RPS)])
            for r in range(K):
                onesb[r, :] = jnp.ones((16,), jnp.float32)
        h_view = h_hbm.at[c]
        for g in range(LA):
            pltpu.async_copy(h_view.at[srcb.at[g]], rows[g], gsem[g])
        plsc.subcore_barrier()

        def step(g, b):
            pltpu.make_async_copy(h_view.at[srcb.at[g]],
                                  rows[b], gsem[b]).wait()
            pltpu.async_copy(rows[b], agg_sh.at[dstb.at[g]], ssem[b],
                             add=True)
            if with_deg:
                @pl.when(c == 0)
                def _():
                    pltpu.async_copy(onesb, deg_sh.at[dstb.at[g]], dsem,
                                     add=True)
                    @pl.when(g >= LA)
                    def _():
                        pltpu.make_async_copy(
                            onesb, deg_sh.at[dstb.at[g]], dsem).wait()
            bs = (b - LA) % NB
            @pl.when(g >= LA)
            def _():
                pltpu.make_async_copy(rows[bs], agg_sh.at[dstb.at[g]],
                                      ssem[bs]).wait()
            bg = (b + LA) % NB
            @pl.when(g + LA < CHC)
            def _():
                pltpu.async_copy(h_view.at[srcb.at[jnp.minimum(
                    g + LA, CHC - 1)]], rows[bg], gsem[bg])

        T = (CHC // NB) * NB

        def outer(t, carry):
            for b in range(NB):
                step(t * NB + b, b)
            return carry

        lax.fori_loop(0, T // NB, outer, 0)
        for g in range(T, CHC):
            step(g, g % NB)
        for g in range(CHC - LA, CHC):
            pltpu.make_async_copy(rows[g % NB], agg_sh.at[dstb.at[0]],
                                  ssem[g % NB]).wait()
            if with_deg:
                @pl.when(c == 0)
                def _():
                    pltpu.make_async_copy(onesb, deg_sh.at[dstb.at[0]],
                                          dsem).wait()
        plsc.subcore_barrier()
        pltpu.sync_copy(agg_sh.at[pl.ds(r0, RPS)],
                        out_hbm.at[c, pl.ds(r0, RPS)])
        if with_deg:
            @pl.when(c == 0)
            def _():
                pltpu.sync_copy(deg_sh.at[pl.ds(r0, RPS)],
                                deg_hbm.at[pl.ds(r0, RPS)])

    return pl.kernel(body, out_type=tuple(out_type), mesh=mesh,
                     scratch_types=scratch,
                     compiler_params=pltpu.CompilerParams(
                         use_tc_tiling_on_sc=False))


def _full(shape):
    return pl.BlockSpec(shape, lambda i: tuple(0 for _ in shape))


def _split_cols(res, hl_ref, dh):
    hl_ref[0] = res[:, :dh]
    hl_ref[1] = res[:, dh:]


def _tc_transform(x, wlT, wrT, b):
    d_in = wlT.shape[0]

    def body(x_ref, wl_ref, wr_ref, b_ref, hl_ref, sf_ref):
        xb = x_ref[...]
        _split_cols(jnp.dot(xb, wl_ref[...],
                            preferred_element_type=jnp.float32), hl_ref, 64)
        sf_ref[...] = jnp.dot(xb, wr_ref[...],
                              preferred_element_type=jnp.float32) + b_ref[...]

    return pl.pallas_call(
        body,
        grid=(G,),
        in_specs=[
            pl.BlockSpec((R, d_in), lambda i: (i, 0)),
            _full((d_in, 128)), _full((d_in, 128)), _full((1, 128)),
        ],
        out_specs=[pl.BlockSpec((2, R, 64), lambda i: (0, i, 0)),
                   pl.BlockSpec((R, 128), lambda i: (i, 0))],
        out_shape=[jax.ShapeDtypeStruct((2, N, 64), jnp.float32),
                   jax.ShapeDtypeStruct((N, 128), jnp.float32)],
    )(x, wlT, wrT, b)


def _inv_deg(deg_ref):
    return 1.0 / jnp.maximum(deg_ref[:, 0:1], 1.0)


def _tc_mid(aggp, deg, sf, wlT, wrT, b, act):
    d_out = wlT.shape[1]
    dh_in = aggp.shape[2]
    dh_out = d_out // 2

    def body(aggp_ref, deg_ref, sf_ref, wl_ref, wr_ref, b_ref,
             hl_ref, sfo_ref):
        agg = jnp.concatenate([aggp_ref[0], aggp_ref[1]], axis=1)
        z = agg * _inv_deg(deg_ref) + sf_ref[...]
        if act == "relu":
            h = jnp.maximum(z, 0.0)
        else:
            h = jnp.where(z > 0, z, jnp.exp(jnp.minimum(z, 0.0)) - 1.0)
        hl = jnp.dot(h, wl_ref[...], preferred_element_type=jnp.float32)
        _split_cols(hl, hl_ref, dh_out)
        sfo_ref[...] = jnp.dot(h, wr_ref[...],
                               preferred_element_type=jnp.float32) + b_ref[...]

    hl_spec = pl.BlockSpec((2, R, dh_out), lambda i: (0, i, 0))
    hl_shape = jax.ShapeDtypeStruct((2, N, dh_out), jnp.float32)

    return pl.pallas_call(
        body,
        grid=(G,),
        in_specs=[
            pl.BlockSpec((NC, R, dh_in), lambda i: (0, i, 0)),
            pl.BlockSpec((R, 16), lambda i: (i, 0)),
            pl.BlockSpec((R, 128), lambda i: (i, 0)),
            _full((128, d_out)), _full((128, d_out)), _full((1, d_out)),
        ],
        out_specs=[hl_spec, pl.BlockSpec((R, d_out), lambda i: (i, 0))],
        out_shape=[hl_shape, jax.ShapeDtypeStruct((N, d_out), jnp.float32)],
    )(aggp, deg, sf, wlT, wrT, b)


def _tc_final(aggp, deg, sf):
    def body(aggp_ref, deg_ref, sf_ref, out_ref):
        agg = jnp.concatenate([aggp_ref[0], aggp_ref[1]], axis=1)
        z = agg * _inv_deg(deg_ref) + sf_ref[...]
        m = jnp.max(z, axis=1, keepdims=True)
        lse = m + jnp.log(jnp.sum(jnp.exp(z - m), axis=1, keepdims=True))
        out_ref[...] = z - lse

    return pl.pallas_call(
        body,
        grid=(G,),
        in_specs=[
            pl.BlockSpec((NC, R, 32), lambda i: (0, i, 0)),
            pl.BlockSpec((R, 16), lambda i: (i, 0)),
            pl.BlockSpec((R, 64), lambda i: (i, 0)),
        ],
        out_specs=pl.BlockSpec((R, 64), lambda i: (i, 0)),
        out_shape=jax.ShapeDtypeStruct((N, 64), jnp.float32),
    )(aggp, deg, sf)


def kernel(x, edge_index, edge_attr, Wl1, Wr1, b1, Wlh, Wrh, bh, Wl2, Wr2, b2):
    del edge_attr
    src_c = edge_index[0].reshape(NS, CHC, K)
    dst_c = edge_index[1].reshape(NS, CHC, K)
    z64 = jnp.zeros((NP8, 64), jnp.float32)
    z32 = jnp.zeros((NP8, 32), jnp.float32)
    z16 = jnp.zeros((NP8, 16), jnp.float32)

    hl1, sf1 = _tc_transform(x, Wl1.T, Wr1.T, b1.reshape(1, -1))
    aggp1, deg = _make_colsplit(64, True)(hl1, src_c, dst_c, z64, z16)
    hl2, sf2 = _tc_mid(aggp1, deg, sf1, Wlh.T, Wrh.T, bh.reshape(1, -1),
                       "relu")
    aggp2 = _make_colsplit(64, False)(hl2, src_c, dst_c, z64, z16)
    if isinstance(aggp2, (tuple, list)):
        aggp2 = aggp2[0]
    hl3, sf3 = _tc_mid(aggp2, deg, sf2, Wl2.T, Wr2.T, b2.reshape(1, -1),
                       "elu")
    aggp3 = _make_colsplit(32, False)(hl3, src_c, dst_c, z32, z16)
    if isinstance(aggp3, (tuple, list)):
        aggp3 = aggp3[0]
    return _tc_final(aggp3, deg, sf3)


# speedup vs baseline: 12.9256x; 1.1001x over previous
import functools

import jax
import jax.numpy as jnp
from jax import lax
from jax.experimental import pallas as pl
from jax.experimental.pallas import tpu as pltpu
from jax.experimental.pallas import tpu_sc as plsc

N = 10000
E = 320000
NC, NS = 2, 16
NW = NC * NS
K = 80
EPC = E // NS
CHC = EPC // K
NP8 = 10008
RPS = 632
NB = 6
LA = 3
R = 1000
G = N // R


def _stripe(s):
    return pl.multiple_of(jnp.minimum(s * RPS, NP8 - RPS), 8)


@functools.cache
def _make_colsplit(DH, with_deg):
    mesh = plsc.VectorSubcoreMesh(core_axis_name="c", subcore_axis_name="s",
                                  num_cores=NC, num_subcores=NS)
    out_type = [jax.ShapeDtypeStruct((NP8, 128), jnp.float32)]
    scratch = [
        pltpu.VMEM((CHC, K), jnp.int32),
        pltpu.VMEM((CHC, K), jnp.int32),
        [pltpu.VMEM((K, DH), jnp.float32)] * NB,
        pltpu.VMEM_SHARED((NP8, DH), jnp.float32),
        [pltpu.SemaphoreType.DMA] * NB,
        [pltpu.SemaphoreType.DMA] * NB,
        pltpu.SemaphoreType.DMA,
    ]
    if with_deg:
        out_type.append(jax.ShapeDtypeStruct((NP8, 16), jnp.float32))
        scratch += [
            pltpu.VMEM((K, 16), jnp.float32),
            pltpu.VMEM_SHARED((NP8, 16), jnp.float32),
        ]

    def body(h_hbm, src_hbm, dst_hbm, z_hbm, z16_hbm, *rest):
        if with_deg:
            (out_hbm, deg_hbm, srcb, dstb, rows, agg_sh, gsem, ssem, dsem,
             onesb, deg_sh) = rest
        else:
            (out_hbm, srcb, dstb, rows, agg_sh, gsem, ssem, dsem) = rest
        c = lax.axis_index("c")
        s = lax.axis_index("s")
        pltpu.sync_copy(src_hbm.at[c * NS + s], srcb)
        pltpu.sync_copy(dst_hbm.at[s], dstb)
        r0 = _stripe(s)
        pltpu.sync_copy(z_hbm.at[pl.ds(r0, RPS)], agg_sh.at[pl.ds(r0, RPS)])
        if with_deg:
            @pl.when(c == 0)
            def _():
                pltpu.sync_copy(z16_hbm.at[pl.ds(r0, RPS)],
                                deg_sh.at[pl.ds(r0, RPS)])
            for r in range(K):
                onesb[r, :] = jnp.ones((16,), jnp.float32)
        for g in range(LA):
            pltpu.async_copy(h_hbm.at[srcb.at[g]], rows[g], gsem[g])
        plsc.subcore_barrier()

        def step(g, b):
            pltpu.make_async_copy(h_hbm.at[srcb.at[g]],
                                  rows[b], gsem[b]).wait()
            pltpu.async_copy(rows[b], agg_sh.at[dstb.at[g]], ssem[b],
                             add=True)
            if with_deg:
                @pl.when(c == 0)
                def _():
                    pltpu.async_copy(onesb, deg_sh.at[dstb.at[g]], dsem,
                                     add=True)
                    @pl.when(g >= LA)
                    def _():
                        pltpu.make_async_copy(
                            onesb, deg_sh.at[dstb.at[g]], dsem).wait()
            bs = (b - LA) % NB
            @pl.when(g >= LA)
            def _():
                pltpu.make_async_copy(rows[bs], agg_sh.at[dstb.at[g]],
                                      ssem[bs]).wait()
            bg = (b + LA) % NB
            @pl.when(g + LA < CHC)
            def _():
                pltpu.async_copy(h_hbm.at[srcb.at[jnp.minimum(
                    g + LA, CHC - 1)]], rows[bg], gsem[bg])

        T = (CHC // NB) * NB

        def outer(t, carry):
            for b in range(NB):
                step(t * NB + b, b)
            return carry

        lax.fori_loop(0, T // NB, outer, 0)
        for g in range(T, CHC):
            step(g, g % NB)
        for g in range(CHC - LA, CHC):
            pltpu.make_async_copy(rows[g % NB], agg_sh.at[dstb.at[0]],
                                  ssem[g % NB]).wait()
            if with_deg:
                @pl.when(c == 0)
                def _():
                    pltpu.make_async_copy(onesb, deg_sh.at[dstb.at[0]],
                                          dsem).wait()
        plsc.subcore_barrier()

        @pl.when(c == 0)
        def _():
            pltpu.sync_copy(agg_sh.at[pl.ds(r0, RPS)],
                            out_hbm.at[pl.ds(r0, RPS), pl.ds(0, DH)])
            if with_deg:
                pltpu.sync_copy(deg_sh.at[pl.ds(r0, RPS)],
                                deg_hbm.at[pl.ds(r0, RPS)])

        @pl.when(c == 1)
        def _():
            pltpu.sync_copy(agg_sh.at[pl.ds(r0, RPS)],
                            out_hbm.at[pl.ds(r0, RPS), pl.ds(DH, DH)])

    return pl.kernel(body, out_type=tuple(out_type), mesh=mesh,
                     scratch_types=scratch,
                     compiler_params=pltpu.CompilerParams(
                         use_tc_tiling_on_sc=False))


def _full(shape):
    return pl.BlockSpec(shape, lambda i: tuple(0 for _ in shape))


def _tc_transform(x, wlT, wrT, b):
    d_in = wlT.shape[0]

    def body(x_ref, wl_ref, wr_ref, b_ref, hl_ref, sf_ref):
        xb = x_ref[...]
        hl_ref[...] = jnp.dot(xb, wl_ref[...],
                              preferred_element_type=jnp.float32)
        sf_ref[...] = jnp.dot(xb, wr_ref[...],
                              preferred_element_type=jnp.float32) + b_ref[...]

    return pl.pallas_call(
        body,
        grid=(G,),
        in_specs=[
            pl.BlockSpec((R, d_in), lambda i: (i, 0)),
            _full((d_in, 128)), _full((d_in, 128)), _full((1, 128)),
        ],
        out_specs=[pl.BlockSpec((R, 128), lambda i: (i, 0)),
                   pl.BlockSpec((R, 128), lambda i: (i, 0))],
        out_shape=[jax.ShapeDtypeStruct((N, 128), jnp.float32),
                   jax.ShapeDtypeStruct((N, 128), jnp.float32)],
    )(x, wlT, wrT, b)


def _inv_deg(deg_ref):
    return 1.0 / jnp.maximum(deg_ref[:, 0:1], 1.0)


def _tc_mid(agg, deg, sf, wlT, wrT, b, act, pad_hl):
    d_out = wlT.shape[1]

    def body(agg_ref, deg_ref, sf_ref, wl_ref, wr_ref, b_ref,
             hl_ref, sfo_ref):
        z = agg_ref[...] * _inv_deg(deg_ref) + sf_ref[...]
        if act == "relu":
            h = jnp.maximum(z, 0.0)
        else:
            h = jnp.where(z > 0, z, jnp.exp(jnp.minimum(z, 0.0)) - 1.0)
        hl = jnp.dot(h, wl_ref[...], preferred_element_type=jnp.float32)
        if pad_hl:
            hl = jnp.concatenate(
                [hl, jnp.zeros((R, 128 - d_out), jnp.float32)], axis=1)
        hl_ref[...] = hl
        sfo_ref[...] = jnp.dot(h, wr_ref[...],
                               preferred_element_type=jnp.float32) + b_ref[...]

    return pl.pallas_call(
        body,
        grid=(G,),
        in_specs=[
            pl.BlockSpec((R, 128), lambda i: (i, 0)),
            pl.BlockSpec((R, 16), lambda i: (i, 0)),
            pl.BlockSpec((R, 128), lambda i: (i, 0)),
            _full((128, d_out)), _full((128, d_out)), _full((1, d_out)),
        ],
        out_specs=[pl.BlockSpec((R, 128), lambda i: (i, 0)),
                   pl.BlockSpec((R, d_out), lambda i: (i, 0))],
        out_shape=[jax.ShapeDtypeStruct((N, 128), jnp.float32),
                   jax.ShapeDtypeStruct((N, d_out), jnp.float32)],
    )(agg, deg, sf, wlT, wrT, b)


def _tc_final(agg, deg, sf):
    def body(agg_ref, deg_ref, sf_ref, out_ref):
        z = agg_ref[:, :64] * _inv_deg(deg_ref) + sf_ref[...]
        m = jnp.max(z, axis=1, keepdims=True)
        lse = m + jnp.log(jnp.sum(jnp.exp(z - m), axis=1, keepdims=True))
        out_ref[...] = z - lse

    return pl.pallas_call(
        body,
        grid=(G,),
        in_specs=[
            pl.BlockSpec((R, 128), lambda i: (i, 0)),
            pl.BlockSpec((R, 16), lambda i: (i, 0)),
            pl.BlockSpec((R, 64), lambda i: (i, 0)),
        ],
        out_specs=pl.BlockSpec((R, 64), lambda i: (i, 0)),
        out_shape=jax.ShapeDtypeStruct((N, 64), jnp.float32),
    )(agg, deg, sf)


def kernel(x, edge_index, edge_attr, Wl1, Wr1, b1, Wlh, Wrh, bh, Wl2, Wr2, b2):
    del edge_attr
    src = edge_index[0]
    dst = edge_index[1]
    s2 = src * 2
    src2 = jnp.stack([s2, s2 + 1]).reshape(NC * NS, CHC, K)
    s4 = src * 4
    src4 = jnp.stack([s4, s4 + 1]).reshape(NC * NS, CHC, K)
    dst_c = dst.reshape(NS, CHC, K)
    z64 = jnp.zeros((NP8, 64), jnp.float32)
    z32 = jnp.zeros((NP8, 32), jnp.float32)
    z16 = jnp.zeros((NP8, 16), jnp.float32)

    hl1, sf1 = _tc_transform(x, Wl1.T, Wr1.T, b1.reshape(1, -1))
    agg1, deg = _make_colsplit(64, True)(
        hl1.reshape(NC * N, 64), src2, dst_c, z64, z16)
    hl2, sf2 = _tc_mid(agg1, deg, sf1, Wlh.T, Wrh.T, bh.reshape(1, -1),
                       "relu", False)
    agg2 = _make_colsplit(64, False)(
        hl2.reshape(NC * N, 64), src2, dst_c, z64, z16)
    if isinstance(agg2, (tuple, list)):
        agg2 = agg2[0]
    hl3, sf3 = _tc_mid(agg2, deg, sf2, Wl2.T, Wr2.T, b2.reshape(1, -1),
                       "elu", True)
    agg3 = _make_colsplit(32, False)(
        hl3.reshape(4 * N, 32), src4, dst_c, z32, z16)
    if isinstance(agg3, (tuple, list)):
        agg3 = agg3[0]
    return _tc_final(agg3, deg, sf3)


# speedup vs baseline: 12.9436x; 1.0014x over previous
import functools

import jax
import jax.numpy as jnp
from jax import lax
from jax.experimental import pallas as pl
from jax.experimental.pallas import tpu as pltpu
from jax.experimental.pallas import tpu_sc as plsc

N = 10000
E = 320000
NC, NS = 2, 16
NW = NC * NS
K = 80
EPC = E // NS
CHC = EPC // K
NP8 = 10008
RPS = 632
NB = 6
LA = 3
R = 1000
G = N // R


def _stripe(s):
    return pl.multiple_of(jnp.minimum(s * RPS, NP8 - RPS), 8)


@functools.cache
def _make_colsplit(DH, with_deg):
    mesh = plsc.VectorSubcoreMesh(core_axis_name="c", subcore_axis_name="s",
                                  num_cores=NC, num_subcores=NS)
    out_type = [jax.ShapeDtypeStruct((NP8, 128), jnp.float32)]
    scratch = [
        pltpu.VMEM((CHC, K), jnp.int32),
        pltpu.VMEM((CHC, K), jnp.int32),
        [pltpu.VMEM((K, DH), jnp.float32)] * NB,
        pltpu.VMEM_SHARED((NP8, DH), jnp.float32),
        [pltpu.SemaphoreType.DMA] * NB,
        [pltpu.SemaphoreType.DMA] * NB,
        pltpu.SemaphoreType.DMA,
    ]
    if with_deg:
        out_type.append(jax.ShapeDtypeStruct((NP8, 32), jnp.float32))
        scratch += [
            pltpu.VMEM((K, 16), jnp.float32),
            pltpu.VMEM_SHARED((NP8, 16), jnp.float32),
        ]

    def body(h_hbm, src_hbm, dst_hbm, z_hbm, z16_hbm, *rest):
        if with_deg:
            (out_hbm, deg_hbm, srcb, dstb, rows, agg_sh, gsem, ssem, dsem,
             onesb, deg_sh) = rest
        else:
            (out_hbm, srcb, dstb, rows, agg_sh, gsem, ssem, dsem) = rest
        c = lax.axis_index("c")
        s = lax.axis_index("s")
        pltpu.sync_copy(src_hbm.at[c * NS + s], srcb)
        pltpu.sync_copy(dst_hbm.at[s], dstb)
        r0 = _stripe(s)
        pltpu.sync_copy(z_hbm.at[pl.ds(r0, RPS)], agg_sh.at[pl.ds(r0, RPS)])
        if with_deg:
            pltpu.sync_copy(z16_hbm.at[pl.ds(r0, RPS)],
                            deg_sh.at[pl.ds(r0, RPS)])
            for r in range(K):
                onesb[r, :] = jnp.ones((16,), jnp.float32)
        for g in range(LA):
            pltpu.async_copy(h_hbm.at[srcb.at[g]], rows[g], gsem[g])
        plsc.subcore_barrier()

        def step(g, b):
            pltpu.make_async_copy(h_hbm.at[srcb.at[g]],
                                  rows[b], gsem[b]).wait()
            pltpu.async_copy(rows[b], agg_sh.at[dstb.at[g]], ssem[b],
                             add=True)
            if with_deg:
                @pl.when((g % 2) == c)
                def _():
                    pltpu.async_copy(onesb, deg_sh.at[dstb.at[g]], dsem,
                                     add=True)
                    @pl.when(g >= 4)
                    def _():
                        pltpu.make_async_copy(
                            onesb, deg_sh.at[dstb.at[g]], dsem).wait()
            bs = (b - LA) % NB
            @pl.when(g >= LA)
            def _():
                pltpu.make_async_copy(rows[bs], agg_sh.at[dstb.at[g]],
                                      ssem[bs]).wait()
            bg = (b + LA) % NB
            @pl.when(g + LA < CHC)
            def _():
                pltpu.async_copy(h_hbm.at[srcb.at[jnp.minimum(
                    g + LA, CHC - 1)]], rows[bg], gsem[bg])

        T = (CHC // NB) * NB

        def outer(t, carry):
            for b in range(NB):
                step(t * NB + b, b)
            return carry

        lax.fori_loop(0, T // NB, outer, 0)
        for g in range(T, CHC):
            step(g, g % NB)
        for g in range(CHC - LA, CHC):
            pltpu.make_async_copy(rows[g % NB], agg_sh.at[dstb.at[0]],
                                  ssem[g % NB]).wait()
        if with_deg:
            for _ in range(2):
                pltpu.make_async_copy(onesb, deg_sh.at[dstb.at[0]],
                                      dsem).wait()
        plsc.subcore_barrier()

        @pl.when(c == 0)
        def _():
            pltpu.sync_copy(agg_sh.at[pl.ds(r0, RPS)],
                            out_hbm.at[pl.ds(r0, RPS), pl.ds(0, DH)])
            if with_deg:
                pltpu.sync_copy(deg_sh.at[pl.ds(r0, RPS)],
                                deg_hbm.at[pl.ds(r0, RPS), pl.ds(0, 16)])

        @pl.when(c == 1)
        def _():
            pltpu.sync_copy(agg_sh.at[pl.ds(r0, RPS)],
                            out_hbm.at[pl.ds(r0, RPS), pl.ds(DH, DH)])
            if with_deg:
                pltpu.sync_copy(deg_sh.at[pl.ds(r0, RPS)],
                                deg_hbm.at[pl.ds(r0, RPS), pl.ds(16, 16)])

    return pl.kernel(body, out_type=tuple(out_type), mesh=mesh,
                     scratch_types=scratch,
                     compiler_params=pltpu.CompilerParams(
                         use_tc_tiling_on_sc=False))


def _full(shape):
    return pl.BlockSpec(shape, lambda i: tuple(0 for _ in shape))


def _tc_transform(x, wlT, wrT, b):
    d_in = wlT.shape[0]

    def body(x_ref, wl_ref, wr_ref, b_ref, hl_ref, sf_ref):
        xb = x_ref[...]
        hl_ref[...] = jnp.dot(xb, wl_ref[...],
                              preferred_element_type=jnp.float32)
        sf_ref[...] = jnp.dot(xb, wr_ref[...],
                              preferred_element_type=jnp.float32) + b_ref[...]

    return pl.pallas_call(
        body,
        grid=(G,),
        in_specs=[
            pl.BlockSpec((R, d_in), lambda i: (i, 0)),
            _full((d_in, 128)), _full((d_in, 128)), _full((1, 128)),
        ],
        out_specs=[pl.BlockSpec((R, 128), lambda i: (i, 0)),
                   pl.BlockSpec((R, 128), lambda i: (i, 0))],
        out_shape=[jax.ShapeDtypeStruct((N, 128), jnp.float32),
                   jax.ShapeDtypeStruct((N, 128), jnp.float32)],
    )(x, wlT, wrT, b)


def _inv_deg(deg_ref):
    deg = deg_ref[:, 0:1] + deg_ref[:, 16:17]
    return 1.0 / jnp.maximum(deg, 1.0)


def _tc_mid(agg, deg, sf, wlT, wrT, b, act, pad_hl):
    d_out = wlT.shape[1]

    def body(agg_ref, deg_ref, sf_ref, wl_ref, wr_ref, b_ref,
             hl_ref, sfo_ref):
        z = agg_ref[...] * _inv_deg(deg_ref) + sf_ref[...]
        if act == "relu":
            h = jnp.maximum(z, 0.0)
        else:
            h = jnp.where(z > 0, z, jnp.exp(jnp.minimum(z, 0.0)) - 1.0)
        hl = jnp.dot(h, wl_ref[...], preferred_element_type=jnp.float32)
        if pad_hl:
            hl = jnp.concatenate(
                [hl, jnp.zeros((R, 128 - d_out), jnp.float32)], axis=1)
        hl_ref[...] = hl
        sfo_ref[...] = jnp.dot(h, wr_ref[...],
                               preferred_element_type=jnp.float32) + b_ref[...]

    return pl.pallas_call(
        body,
        grid=(G,),
        in_specs=[
            pl.BlockSpec((R, 128), lambda i: (i, 0)),
            pl.BlockSpec((R, 32), lambda i: (i, 0)),
            pl.BlockSpec((R, 128), lambda i: (i, 0)),
            _full((128, d_out)), _full((128, d_out)), _full((1, d_out)),
        ],
        out_specs=[pl.BlockSpec((R, 128), lambda i: (i, 0)),
                   pl.BlockSpec((R, d_out), lambda i: (i, 0))],
        out_shape=[jax.ShapeDtypeStruct((N, 128), jnp.float32),
                   jax.ShapeDtypeStruct((N, d_out), jnp.float32)],
    )(agg, deg, sf, wlT, wrT, b)


def _tc_final(agg, deg, sf):
    def body(agg_ref, deg_ref, sf_ref, out_ref):
        z = agg_ref[:, :64] * _inv_deg(deg_ref) + sf_ref[...]
        m = jnp.max(z, axis=1, keepdims=True)
        lse = m + jnp.log(jnp.sum(jnp.exp(z - m), axis=1, keepdims=True))
        out_ref[...] = z - lse

    return pl.pallas_call(
        body,
        grid=(G,),
        in_specs=[
            pl.BlockSpec((R, 128), lambda i: (i, 0)),
            pl.BlockSpec((R, 32), lambda i: (i, 0)),
            pl.BlockSpec((R, 64), lambda i: (i, 0)),
        ],
        out_specs=pl.BlockSpec((R, 64), lambda i: (i, 0)),
        out_shape=jax.ShapeDtypeStruct((N, 64), jnp.float32),
    )(agg, deg, sf)


def kernel(x, edge_index, edge_attr, Wl1, Wr1, b1, Wlh, Wrh, bh, Wl2, Wr2, b2):
    del edge_attr
    src = edge_index[0]
    dst = edge_index[1]
    s2 = src * 2
    src2 = jnp.stack([s2, s2 + 1]).reshape(NC * NS, CHC, K)
    s4 = src * 4
    src4 = jnp.stack([s4, s4 + 1]).reshape(NC * NS, CHC, K)
    dst_c = dst.reshape(NS, CHC, K)
    z64 = jnp.zeros((NP8, 64), jnp.float32)
    z32 = jnp.zeros((NP8, 32), jnp.float32)
    z16 = jnp.zeros((NP8, 16), jnp.float32)

    hl1, sf1 = _tc_transform(x, Wl1.T, Wr1.T, b1.reshape(1, -1))
    agg1, deg = _make_colsplit(64, True)(
        hl1.reshape(NC * N, 64), src2, dst_c, z64, z16)
    hl2, sf2 = _tc_mid(agg1, deg, sf1, Wlh.T, Wrh.T, bh.reshape(1, -1),
                       "relu", False)
    agg2 = _make_colsplit(64, False)(
        hl2.reshape(NC * N, 64), src2, dst_c, z64, z16)
    if isinstance(agg2, (tuple, list)):
        agg2 = agg2[0]
    hl3, sf3 = _tc_mid(agg2, deg, sf2, Wl2.T, Wr2.T, b2.reshape(1, -1),
                       "elu", True)
    agg3 = _make_colsplit(32, False)(
        hl3.reshape(4 * N, 32), src4, dst_c, z32, z16)
    if isinstance(agg3, (tuple, list)):
        agg3 = agg3[0]
    return _tc_final(agg3, deg, sf3)
